# baseline (speedup 1.0000x reference)
import functools

import jax
import jax.numpy as jnp
from jax import lax
from jax.experimental import pallas as pl
from jax.experimental.pallas import tpu as pltpu
from jax.experimental.pallas import tpu_sc as plsc

_U, _I, _B, _D = 10000, 10000, 10000, 256
_E = 160000
_EPS = 1e-8

_NR = 20480
_NRB = 10240
_BLK = 1024
_EB = 125
_HW = 16

_MESH = plsc.VectorSubcoreMesh(core_axis_name="c", subcore_axis_name="s")
_SC_PARAMS = pltpu.CompilerParams(use_tc_tiling_on_sc=False)


def _hist_sc(dstb_ui, dstb_ub, dstb_bi):
    rounds = [(dstb_ui.shape[0], _NR, 0), (dstb_ub.shape[0], _NR, 1),
              (dstb_bi.shape[0], _NRB, None)]

    @functools.partial(
        pl.kernel,
        out_type=[jax.ShapeDtypeStruct((4 * _NR, _HW), jnp.float32),
                  jax.ShapeDtypeStruct((2 * _NRB, _HW), jnp.float32)],
        mesh=_MESH,
        compiler_params=_SC_PARAMS,
        scratch_types=[
            pltpu.VMEM((80, _EB), jnp.int32),
            pltpu.VMEM((_EB, _HW), jnp.float32),
            pltpu.VMEM((128, _HW), jnp.float32),
            pltpu.VMEM_SHARED((_NR, _HW), jnp.float32),
            pltpu.SemaphoreType.DMA,
        ],
    )
    def k(ui_hbm, ub_hbm, bi_hbm, degs_hbm, cnt_hbm,
          dstv, onesb, zbuf, acc, sem):
        cid = lax.axis_index("c")
        sid = lax.axis_index("s")
        wid = sid * 2 + cid
        ones16 = jnp.ones((16,), jnp.float32)
        zero16 = jnp.zeros((16,), jnp.float32)

        def obody(i, _):
            onesb[i, pl.ds(0, 16)] = ones16
            return 0
        lax.fori_loop(0, _EB, obody, 0)

        def zbody(i, _):
            zbuf[i, pl.ds(0, 16)] = zero16
            return 0
        lax.fori_loop(0, 128, zbody, 0)

        for rnd, (nblk, npad, lv) in enumerate(rounds):
            src_hbm = (ui_hbm, ub_hbm, bi_hbm)[rnd]
            bpt = nblk // 32
            stripe = npad // 16
            pltpu.sync_copy(src_hbm.at[pl.ds(wid * bpt, bpt)],
                            dstv.at[pl.ds(0, bpt)])
            for t in range(stripe // 128):
                pltpu.sync_copy(zbuf,
                                acc.at[pl.ds(sid * stripe + t * 128, 128)])
            plsc.subcore_barrier()

            def body(n, _):
                for j in range(4):
                    pltpu.async_copy(onesb, acc.at[dstv.at[n * 4 + j]], sem,
                                     add=True)
                for j in range(4):
                    pltpu.make_async_copy(onesb, acc.at[dstv.at[0]],
                                          sem).wait()
                return 0
            lax.fori_loop(0, bpt // 4, body, 0)
            plsc.subcore_barrier()
            if lv is None:
                pltpu.sync_copy(
                    acc.at[pl.ds(sid * stripe, stripe)],
                    cnt_hbm.at[pl.ds(cid * npad + sid * stripe, stripe)])
            else:
                pltpu.sync_copy(
                    acc.at[pl.ds(sid * stripe, stripe)],
                    degs_hbm.at[pl.ds((2 * lv + cid) * npad + sid * stripe,
                                      stripe)])
            plsc.subcore_barrier()

    return k(dstb_ui, dstb_ub, dstb_bi)


def _scatter_sc(table, srcg, dstb, npad_out, n_lv):
    nblk = dstb.shape[0] // n_lv
    bpt = nblk // 16
    SG = 8
    sgrps = bpt // SG
    stripe = npad_out // 16
    zcop = stripe // 64

    @functools.partial(
        pl.kernel,
        out_type=jax.ShapeDtypeStruct((n_lv * 4 * npad_out, 64), jnp.float32),
        mesh=_MESH,
        compiler_params=_SC_PARAMS,
        scratch_types=[
            pltpu.VMEM((3, SG, _EB), jnp.int32),
            pltpu.VMEM((3, SG, _EB), jnp.int32),
            pltpu.VMEM((4, _EB, 64), jnp.float32),
            pltpu.VMEM((64, 64), jnp.float32),
            pltpu.VMEM_SHARED((npad_out, 64), jnp.float32),
            pltpu.SemaphoreType.DMA,
            pltpu.SemaphoreType.DMA,
            pltpu.SemaphoreType.DMA,
        ],
    )
    def k(tab_hbm, srcg_hbm, dstb_hbm, out_hbm,
          srcv, dstv, rowsb, zbuf, acc, sem_g, sem_s, sem_i):
        cid = lax.axis_index("c")
        sid = lax.axis_index("s")
        zero16 = jnp.zeros((16,), jnp.float32)

        def zbody(i, _):
            r = lax.shift_right_logical(i, 2)
            c = lax.bitwise_and(i, 3)
            zbuf[r, pl.ds(c * 16, 16)] = zero16
            return 0
        lax.fori_loop(0, 256, zbody, 0)

        for lv in range(n_lv):
            for kk in range(2):
                chunk = 2 * cid + kk
                sb0 = (4 * lv + chunk) * nblk + sid * bpt
                db0 = lv * nblk + sid * bpt

                def fire_is(s, par):
                    pltpu.async_copy(srcg_hbm.at[pl.ds(sb0 + s * SG, SG)],
                                     srcv.at[par], sem_i)
                    pltpu.async_copy(dstb_hbm.at[pl.ds(db0 + s * SG, SG)],
                                     dstv.at[par], sem_i)

                def drain_is(par):
                    for _ in range(2):
                        pltpu.make_async_copy(dstb_hbm.at[pl.ds(db0, SG)],
                                              dstv.at[par], sem_i).wait()

                def fire_g(par, r):
                    pltpu.async_copy(tab_hbm.at[srcv.at[par, r]],
                                     rowsb.at[r % 4], sem_g)

                def drain_g(r):
                    pltpu.make_async_copy(tab_hbm.at[srcv.at[0, 0]],
                                          rowsb.at[r % 4], sem_g).wait()

                def fire_s(par, r):
                    pltpu.async_copy(rowsb.at[r % 4],
                                     acc.at[dstv.at[par, r]],
                                     sem_s, add=True)

                def drain_s(r):
                    pltpu.make_async_copy(rowsb.at[r % 4],
                                          acc.at[dstv.at[0, 0]],
                                          sem_s).wait()

                def steady_rows(par, pp, first):
                    for r in range(SG):
                        if not first or r >= 4:
                            drain_s(r % 4)
                        fire_g(par, r)
                        if first and r < 2:
                            continue
                        if r < 2:
                            drain_g((r - 2) % 4)
                            fire_s(pp, SG + r - 2)
                        else:
                            drain_g(r - 2)
                            fire_s(par, r - 2)

                for t in range(zcop):
                    pltpu.sync_copy(
                        zbuf, acc.at[pl.ds(sid * stripe + t * 64, 64)])
                plsc.subcore_barrier()

                fire_is(0, 0)
                drain_is(0)
                fire_is(1, 1)
                steady_rows(0, 0, True)

                def body(s, _):
                    par = lax.rem(s, 3)
                    pp = lax.rem(s + 2, 3)
                    pn = lax.rem(s + 1, 3)
                    drain_is(par)
                    fire_is(s + 1, pn)
                    steady_rows(par, pp, False)
                    return 0
                lax.fori_loop(1, sgrps - 1, body, 0)

                pe = (sgrps - 1) % 3
                drain_is(pe)
                steady_rows(pe, (sgrps - 2) % 3, False)
                drain_g(2)
                fire_s(pe, SG - 2)
                drain_g(3)
                fire_s(pe, SG - 1)
                for r in range(4):
                    drain_s(r)

                plsc.subcore_barrier()
                pltpu.sync_copy(
                    acc.at[pl.ds(sid * stripe, stripe)],
                    out_hbm.at[pl.ds((4 * lv + chunk) * npad_out
                                     + sid * stripe, stripe)])
                plsc.subcore_barrier()

    return k(table, srcg, dstb)


def _prep_tc(degs, x2):
    def body(d_ref, x_ref, a_ref, hp_ref):
        d = d_ref[0, 0, :, 0:1] + d_ref[0, 1, :, 0:1] + 1.0
        a = 1.0 / (jnp.sqrt(d) + _EPS)
        a_ref[0] = a
        hp = a * x_ref[0]
        for c in range(4):
            hp_ref[0, c] = hp[:, c * 64:(c + 1) * 64]

    return pl.pallas_call(
        body,
        grid=(2, _NR // _BLK),
        in_specs=[pl.BlockSpec((1, 2, _BLK, _HW), lambda l, i: (l, 0, i, 0)),
                  pl.BlockSpec((1, _BLK, _D), lambda l, i: (l, i, 0))],
        out_specs=[pl.BlockSpec((1, _BLK, 1), lambda l, i: (l, i, 0)),
                   pl.BlockSpec((1, 4, _BLK, 64), lambda l, i: (l, 0, i, 0))],
        out_shape=[jax.ShapeDtypeStruct((2, _NR, 1), jnp.float32),
                   jax.ShapeDtypeStruct((2, 4, _NR, 64), jnp.float32)],
    )(degs, x2)


def _layer1_tc(s4, a2, x2, Ws):
    def body(s_ref, a_ref, h_ref, w_ref, h1_ref, hp_ref):
        aa = a_ref[0]
        s = jnp.concatenate([s_ref[0, c] for c in range(4)], axis=1)
        m = aa * (s + aa * h_ref[0])
        h1 = jnp.tanh(jnp.dot(m, w_ref[0],
                              preferred_element_type=jnp.float32))
        h1_ref[0] = h1
        hp = aa * h1
        for c in range(4):
            hp_ref[0, c] = hp[:, c * 64:(c + 1) * 64]

    return pl.pallas_call(
        body,
        grid=(2, _NR // _BLK),
        in_specs=[pl.BlockSpec((1, 4, _BLK, 64), lambda l, i: (l, 0, i, 0)),
                  pl.BlockSpec((1, _BLK, 1), lambda l, i: (l, i, 0)),
                  pl.BlockSpec((1, _BLK, _D), lambda l, i: (l, i, 0)),
                  pl.BlockSpec((1, _D, _D), lambda l, i: (l, 0, 0))],
        out_specs=[pl.BlockSpec((1, _BLK, _D), lambda l, i: (l, i, 0)),
                   pl.BlockSpec((1, 4, _BLK, 64), lambda l, i: (l, 0, i, 0))],
        out_shape=[jax.ShapeDtypeStruct((2, _NR, _D), jnp.float32),
                   jax.ShapeDtypeStruct((2, 4, _NR, 64), jnp.float32)],
    )(s4, a2, x2, Ws)


def _layer2_tc(s4, a2, h1s, x2, Ws):
    def body(s_ref, a_ref, h1_ref, x_ref, w_ref, o_ref, oc_ref):
        aa = a_ref[0]
        h1 = h1_ref[0]
        s = jnp.concatenate([s_ref[0, c] for c in range(4)], axis=1)
        m = aa * (s + aa * h1)
        h2 = jnp.tanh(jnp.dot(m, w_ref[0],
                              preferred_element_type=jnp.float32))
        o = (x_ref[0] + h1 + h2) * (1.0 / 3.0)
        o_ref[0] = o
        for c in range(4):
            oc_ref[0, c] = o[:, c * 64:(c + 1) * 64]

    return pl.pallas_call(
        body,
        grid=(2, _NR // _BLK),
        in_specs=[pl.BlockSpec((1, 4, _BLK, 64), lambda l, i: (l, 0, i, 0)),
                  pl.BlockSpec((1, _BLK, 1), lambda l, i: (l, i, 0)),
                  pl.BlockSpec((1, _BLK, _D), lambda l, i: (l, i, 0)),
                  pl.BlockSpec((1, _BLK, _D), lambda l, i: (l, i, 0)),
                  pl.BlockSpec((1, _D, _D), lambda l, i: (l, 0, 0))],
        out_specs=[pl.BlockSpec((1, _BLK, _D), lambda l, i: (l, i, 0)),
                   pl.BlockSpec((1, 4, _BLK, 64), lambda l, i: (l, 0, i, 0))],
        out_shape=[jax.ShapeDtypeStruct((2, _NR, _D), jnp.float32),
                   jax.ShapeDtypeStruct((2, 4, _NR, 64), jnp.float32)],
    )(s4, a2, h1s, x2, Ws)


def _gate_tc(il, il4, cnt, bl, feat, gate_W, gate_b):
    nr = bl.shape[0]
    chunked = il4 is not None

    def body(*refs):
        if chunked:
            il_ref, cnt_ref, bl_ref, f_ref, w_ref, b_ref, o_ref = refs
            cntv = cnt_ref[0, :, 0:1] + cnt_ref[1, :, 0:1]
            ilv = jnp.concatenate([il_ref[c] for c in range(4)], axis=1)
            ilv = ilv / (cntv + _EPS)
        else:
            il_ref, bl_ref, f_ref, w_ref, b_ref, o_ref = refs
            ilv = il_ref[...]
        blv = bl_ref[...]
        g = (jnp.dot(ilv, w_ref[0:_D, :], preferred_element_type=jnp.float32)
             + jnp.dot(blv, w_ref[_D:2 * _D, :],
                       preferred_element_type=jnp.float32)
             + jnp.dot(f_ref[...], w_ref[2 * _D:3 * _D, :],
                       preferred_element_type=jnp.float32)
             + b_ref[...])
        m = jnp.max(g, axis=1, keepdims=True)
        e = jnp.exp(g - m)
        w0 = e[:, 0:1] / (e[:, 0:1] + e[:, 1:2])
        o_ref[...] = w0 * ilv + (1.0 - w0) * blv

    in_specs = []
    args = []
    if chunked:
        in_specs += [pl.BlockSpec((4, _BLK, 64), lambda i: (0, i, 0)),
                     pl.BlockSpec((2, _BLK, _HW), lambda i: (0, i, 0))]
        args += [il4, cnt]
    else:
        in_specs.append(pl.BlockSpec((_BLK, _D), lambda i: (i, 0)))
        args.append(il)
    in_specs += [pl.BlockSpec((_BLK, _D), lambda i: (i, 0)),
                 pl.BlockSpec((_BLK, _D), lambda i: (i, 0)),
                 pl.BlockSpec((3 * _D, 2), lambda i: (0, 0)),
                 pl.BlockSpec((1, 2), lambda i: (0, 0))]
    args += [bl, feat, gate_W, gate_b.reshape(1, 2)]

    return pl.pallas_call(
        body,
        grid=(nr // _BLK,),
        in_specs=in_specs,
        out_specs=pl.BlockSpec((_BLK, _D), lambda i: (i, 0)),
        out_shape=jax.ShapeDtypeStruct((nr, _D), jnp.float32),
    )(*args)


def _pad_rows(x, nr):
    return jnp.pad(x, ((0, nr - x.shape[0]), (0, 0)))


def kernel(ui_edge_index, ub_edge_index, bi_edge_index, users_feature,
           items_feature, bundles_feature, W1_item, W2_item, W1_bundle,
           W2_bundle, gate_W, gate_b):
    offs4 = jnp.arange(4, dtype=jnp.int32) * _NR

    def edges(ei):
        src = jnp.concatenate([ei[0], ei[1] + _U])
        dst = jnp.concatenate([ei[1] + _U, ei[0]])
        return src, dst

    src_ui, dst_ui = edges(ui_edge_index)
    src_ub, dst_ub = edges(ub_edge_index)
    srcg = jnp.concatenate(
        [(src_ui[None, :] + offs4[:, None]).reshape(-1, _EB),
         (src_ub[None, :] + (offs4 + 4 * _NR)[:, None]).reshape(-1, _EB)])
    dstb = jnp.concatenate([dst_ui.reshape(-1, _EB),
                            dst_ub.reshape(-1, _EB)])

    b_idx, i_idx = bi_edge_index[0], bi_edge_index[1]
    psrcg = ((i_idx + _U)[None, :] + offs4[:, None]).reshape(-1, _EB)
    pdstb = b_idx.reshape(-1, _EB)

    degs, cnt = _hist_sc(dst_ui.reshape(-1, _EB), dst_ub.reshape(-1, _EB),
                         pdstb)
    degs = degs.reshape(2, 2, _NR, _HW)
    cnt = cnt.reshape(2, _NRB, _HW)

    x2 = jnp.stack([
        _pad_rows(jnp.concatenate([users_feature, items_feature], axis=0),
                  _NR),
        _pad_rows(jnp.concatenate([users_feature, bundles_feature], axis=0),
                  _NR)])
    W1s = jnp.stack([W1_item, W1_bundle])
    W2s = jnp.stack([W2_item, W2_bundle])

    a2, hp0 = _prep_tc(degs, x2)
    s1 = _scatter_sc(hp0.reshape(8 * _NR, 64), srcg, dstb, _NR, 2)
    h1s, hp1 = _layer1_tc(s1.reshape(2, 4, _NR, 64), a2, x2, W1s)
    s2 = _scatter_sc(hp1.reshape(8 * _NR, 64), srcg, dstb, _NR, 2)
    outs, chunk2 = _layer2_tc(s2.reshape(2, 4, _NR, 64), a2, h1s, x2, W2s)

    pooled4 = _scatter_sc(chunk2.reshape(8 * _NR, 64), psrcg, pdstb, _NRB, 1)

    users_il = _pad_rows(outs[0, :_U], _NRB)
    users_bl = _pad_rows(outs[1, :_U], _NRB)
    bundles_bl = _pad_rows(outs[1, _U:_U + _B], _NRB)
    uf = _pad_rows(users_feature, _NRB)
    bf = _pad_rows(bundles_feature, _NRB)

    users_out = _gate_tc(users_il, None, None, users_bl, uf, gate_W, gate_b)
    bundles_out = _gate_tc(None, pooled4.reshape(4, _NRB, 64), cnt,
                           bundles_bl, bf, gate_W, gate_b)
    return jnp.concatenate([users_out[:_U], bundles_out[:_B]], axis=0)


# speedup vs baseline: 1.1947x; 1.1947x over previous
import functools

import jax
import jax.numpy as jnp
from jax import lax
from jax.experimental import pallas as pl
from jax.experimental.pallas import tpu as pltpu
from jax.experimental.pallas import tpu_sc as plsc

_U, _I, _B, _D = 10000, 10000, 10000, 256
_E = 160000
_EPS = 1e-8

_NR = 20480
_NRB = 10240
_BLK = 1024
_EB = 125
_HW = 16

_MESH = plsc.VectorSubcoreMesh(core_axis_name="c", subcore_axis_name="s")
_SC_PARAMS = pltpu.CompilerParams(use_tc_tiling_on_sc=False)


def _hist_sc(dstb_ui, dstb_ub, dstb_bi):
    rounds = [(dstb_ui.shape[0], _NR, 0), (dstb_ub.shape[0], _NR, 1),
              (dstb_bi.shape[0], _NRB, None)]

    @functools.partial(
        pl.kernel,
        out_type=[jax.ShapeDtypeStruct((4 * _NR, _HW), jnp.float32),
                  jax.ShapeDtypeStruct((2 * _NRB, _HW), jnp.float32)],
        mesh=_MESH,
        compiler_params=_SC_PARAMS,
        scratch_types=[
            pltpu.VMEM((80, _EB), jnp.int32),
            pltpu.VMEM((_EB, _HW), jnp.float32),
            pltpu.VMEM((128, _HW), jnp.float32),
            pltpu.VMEM_SHARED((_NR, _HW), jnp.float32),
            pltpu.SemaphoreType.DMA,
        ],
    )
    def k(ui_hbm, ub_hbm, bi_hbm, degs_hbm, cnt_hbm,
          dstv, onesb, zbuf, acc, sem):
        cid = lax.axis_index("c")
        sid = lax.axis_index("s")
        wid = sid * 2 + cid
        ones16 = jnp.ones((16,), jnp.float32)
        zero16 = jnp.zeros((16,), jnp.float32)

        def obody(i, _):
            onesb[i, pl.ds(0, 16)] = ones16
            return 0
        lax.fori_loop(0, _EB, obody, 0)

        def zbody(i, _):
            zbuf[i, pl.ds(0, 16)] = zero16
            return 0
        lax.fori_loop(0, 128, zbody, 0)

        for rnd, (nblk, npad, lv) in enumerate(rounds):
            src_hbm = (ui_hbm, ub_hbm, bi_hbm)[rnd]
            bpt = nblk // 32
            stripe = npad // 16
            pltpu.sync_copy(src_hbm.at[pl.ds(wid * bpt, bpt)],
                            dstv.at[pl.ds(0, bpt)])
            for t in range(stripe // 128):
                pltpu.sync_copy(zbuf,
                                acc.at[pl.ds(sid * stripe + t * 128, 128)])
            plsc.subcore_barrier()

            def body(n, _):
                for j in range(4):
                    pltpu.async_copy(onesb, acc.at[dstv.at[n * 4 + j]], sem,
                                     add=True)
                for j in range(4):
                    pltpu.make_async_copy(onesb, acc.at[dstv.at[0]],
                                          sem).wait()
                return 0
            lax.fori_loop(0, bpt // 4, body, 0)
            plsc.subcore_barrier()
            if lv is None:
                pltpu.sync_copy(
                    acc.at[pl.ds(sid * stripe, stripe)],
                    cnt_hbm.at[pl.ds(cid * npad + sid * stripe, stripe)])
            else:
                pltpu.sync_copy(
                    acc.at[pl.ds(sid * stripe, stripe)],
                    degs_hbm.at[pl.ds((2 * lv + cid) * npad + sid * stripe,
                                      stripe)])
            plsc.subcore_barrier()

    return k(dstb_ui, dstb_ub, dstb_bi)


def _scatter_sc(table, srcg, dstb, npad_out, n_lv):
    nblk = dstb.shape[0] // n_lv
    bpt = nblk // 16
    SG = 8
    sgrps = bpt // SG
    stripe = npad_out // 16
    zcop = stripe // 64

    @functools.partial(
        pl.kernel,
        out_type=jax.ShapeDtypeStruct((n_lv * 4 * npad_out, 64), jnp.float32),
        mesh=_MESH,
        compiler_params=_SC_PARAMS,
        scratch_types=[
            pltpu.VMEM((3, SG, _EB), jnp.int32),
            pltpu.VMEM((3, SG, _EB), jnp.int32),
            pltpu.VMEM((4, _EB, 64), jnp.float32),
            pltpu.VMEM((64, 64), jnp.float32),
            pltpu.VMEM_SHARED((npad_out, 64), jnp.float32),
            pltpu.SemaphoreType.DMA,
            pltpu.SemaphoreType.DMA,
            pltpu.SemaphoreType.DMA,
        ],
    )
    def k(tab_hbm, srcg_hbm, dstb_hbm, out_hbm,
          srcv, dstv, rowsb, zbuf, acc, sem_g, sem_s, sem_i):
        cid = lax.axis_index("c")
        sid = lax.axis_index("s")
        zero16 = jnp.zeros((16,), jnp.float32)

        def zbody(i, _):
            r = lax.shift_right_logical(i, 2)
            c = lax.bitwise_and(i, 3)
            zbuf[r, pl.ds(c * 16, 16)] = zero16
            return 0
        lax.fori_loop(0, 256, zbody, 0)

        for lv in range(n_lv):
            for kk in range(2):
                chunk = 2 * cid + kk
                sb0 = (4 * lv + chunk) * nblk + sid * bpt
                db0 = lv * nblk + sid * bpt

                def fire_is(s, par):
                    pltpu.async_copy(srcg_hbm.at[pl.ds(sb0 + s * SG, SG)],
                                     srcv.at[par], sem_i)
                    pltpu.async_copy(dstb_hbm.at[pl.ds(db0 + s * SG, SG)],
                                     dstv.at[par], sem_i)

                def drain_is(par):
                    for _ in range(2):
                        pltpu.make_async_copy(dstb_hbm.at[pl.ds(db0, SG)],
                                              dstv.at[par], sem_i).wait()

                def fire_g(par, r):
                    pltpu.async_copy(tab_hbm.at[srcv.at[par, r]],
                                     rowsb.at[r % 4], sem_g)

                def drain_g(r):
                    pltpu.make_async_copy(tab_hbm.at[srcv.at[0, 0]],
                                          rowsb.at[r % 4], sem_g).wait()

                def fire_s(par, r):
                    pltpu.async_copy(rowsb.at[r % 4],
                                     acc.at[dstv.at[par, r]],
                                     sem_s, add=True)

                def drain_s(r):
                    pltpu.make_async_copy(rowsb.at[r % 4],
                                          acc.at[dstv.at[0, 0]],
                                          sem_s).wait()

                def steady_rows(par, pp, first):
                    for r in range(SG):
                        if not first or r >= 4:
                            drain_s(r % 4)
                        fire_g(par, r)
                        if first and r < 2:
                            continue
                        if r < 2:
                            drain_g((r - 2) % 4)
                            fire_s(pp, SG + r - 2)
                        else:
                            drain_g(r - 2)
                            fire_s(par, r - 2)

                for t in range(zcop):
                    pltpu.sync_copy(
                        zbuf, acc.at[pl.ds(sid * stripe + t * 64, 64)])
                plsc.subcore_barrier()

                fire_is(0, 0)
                drain_is(0)
                fire_is(1, 1)
                steady_rows(0, 0, True)

                def body(s, _):
                    par = lax.rem(s, 3)
                    pp = lax.rem(s + 2, 3)
                    pn = lax.rem(s + 1, 3)
                    drain_is(par)
                    fire_is(s + 1, pn)
                    steady_rows(par, pp, False)
                    return 0
                lax.fori_loop(1, sgrps - 1, body, 0)

                pe = (sgrps - 1) % 3
                drain_is(pe)
                steady_rows(pe, (sgrps - 2) % 3, False)
                drain_g(2)
                fire_s(pe, SG - 2)
                drain_g(3)
                fire_s(pe, SG - 1)
                for r in range(4):
                    drain_s(r)

                plsc.subcore_barrier()
                pltpu.sync_copy(
                    acc.at[pl.ds(sid * stripe, stripe)],
                    out_hbm.at[pl.ds((4 * lv + chunk) * npad_out
                                     + sid * stripe, stripe)])
                plsc.subcore_barrier()

    return k(table, srcg, dstb)


def _prep_tc(deg2, x):
    def body(d_ref, x_ref, a_ref, hp_ref):
        d = d_ref[0, :, 0:1] + d_ref[1, :, 0:1] + 1.0
        a = 1.0 / (jnp.sqrt(d) + _EPS)
        a_ref[...] = a
        hp = a * x_ref[...]
        for c in range(4):
            hp_ref[c] = hp[:, c * 64:(c + 1) * 64]

    return pl.pallas_call(
        body,
        grid=(_NR // _BLK,),
        in_specs=[pl.BlockSpec((2, _BLK, _HW), lambda i: (0, i, 0)),
                  pl.BlockSpec((_BLK, _D), lambda i: (i, 0))],
        out_specs=[pl.BlockSpec((_BLK, 1), lambda i: (i, 0)),
                   pl.BlockSpec((4, _BLK, 64), lambda i: (0, i, 0))],
        out_shape=[jax.ShapeDtypeStruct((_NR, 1), jnp.float32),
                   jax.ShapeDtypeStruct((4, _NR, 64), jnp.float32)],
    )(deg2, x)


def _layer1_tc(s4, a, x, W):
    def body(s_ref, a_ref, h_ref, w_ref, h1_ref, hp_ref):
        aa = a_ref[...]
        s = jnp.concatenate([s_ref[c] for c in range(4)], axis=1)
        m = aa * (s + aa * h_ref[...])
        h1 = jnp.tanh(jnp.dot(m, w_ref[...],
                              preferred_element_type=jnp.float32))
        h1_ref[...] = h1
        hp = aa * h1
        for c in range(4):
            hp_ref[c] = hp[:, c * 64:(c + 1) * 64]

    return pl.pallas_call(
        body,
        grid=(_NR // _BLK,),
        in_specs=[pl.BlockSpec((4, _BLK, 64), lambda i: (0, i, 0)),
                  pl.BlockSpec((_BLK, 1), lambda i: (i, 0)),
                  pl.BlockSpec((_BLK, _D), lambda i: (i, 0)),
                  pl.BlockSpec((_D, _D), lambda i: (0, 0))],
        out_specs=[pl.BlockSpec((_BLK, _D), lambda i: (i, 0)),
                   pl.BlockSpec((4, _BLK, 64), lambda i: (0, i, 0))],
        out_shape=[jax.ShapeDtypeStruct((_NR, _D), jnp.float32),
                   jax.ShapeDtypeStruct((4, _NR, 64), jnp.float32)],
    )(s4, a, x, W)


def _layer2_tc(s4, a, h1, x, W, emit_chunked):
    def body(*refs):
        if emit_chunked:
            s_ref, a_ref, h1_ref, x_ref, w_ref, o_ref, oc_ref = refs
        else:
            s_ref, a_ref, h1_ref, x_ref, w_ref, o_ref = refs
        aa = a_ref[...]
        h1v = h1_ref[...]
        s = jnp.concatenate([s_ref[c] for c in range(4)], axis=1)
        m = aa * (s + aa * h1v)
        h2 = jnp.tanh(jnp.dot(m, w_ref[...],
                              preferred_element_type=jnp.float32))
        o = (x_ref[...] + h1v + h2) * (1.0 / 3.0)
        o_ref[...] = o
        if emit_chunked:
            for c in range(4):
                oc_ref[c] = o[:, c * 64:(c + 1) * 64]

    out_specs = [pl.BlockSpec((_BLK, _D), lambda i: (i, 0))]
    out_shape = [jax.ShapeDtypeStruct((_NR, _D), jnp.float32)]
    if emit_chunked:
        out_specs.append(pl.BlockSpec((4, _BLK, 64), lambda i: (0, i, 0)))
        out_shape.append(jax.ShapeDtypeStruct((4, _NR, 64), jnp.float32))

    return pl.pallas_call(
        body,
        grid=(_NR // _BLK,),
        in_specs=[pl.BlockSpec((4, _BLK, 64), lambda i: (0, i, 0)),
                  pl.BlockSpec((_BLK, 1), lambda i: (i, 0)),
                  pl.BlockSpec((_BLK, _D), lambda i: (i, 0)),
                  pl.BlockSpec((_BLK, _D), lambda i: (i, 0)),
                  pl.BlockSpec((_D, _D), lambda i: (0, 0))],
        out_specs=out_specs,
        out_shape=out_shape,
    )(s4, a, h1, x, W)


def _gate_tc(il, il4, cnt, bl, feat, gate_W, gate_b):
    nr = bl.shape[0]
    chunked = il4 is not None

    def body(*refs):
        if chunked:
            il_ref, cnt_ref, bl_ref, f_ref, w_ref, b_ref, o_ref = refs
            cntv = cnt_ref[0, :, 0:1] + cnt_ref[1, :, 0:1]
            ilv = jnp.concatenate([il_ref[c] for c in range(4)], axis=1)
            ilv = ilv / (cntv + _EPS)
        else:
            il_ref, bl_ref, f_ref, w_ref, b_ref, o_ref = refs
            ilv = il_ref[...]
        blv = bl_ref[...]
        g = (jnp.dot(ilv, w_ref[0:_D, :], preferred_element_type=jnp.float32)
             + jnp.dot(blv, w_ref[_D:2 * _D, :],
                       preferred_element_type=jnp.float32)
             + jnp.dot(f_ref[...], w_ref[2 * _D:3 * _D, :],
                       preferred_element_type=jnp.float32)
             + b_ref[...])
        m = jnp.max(g, axis=1, keepdims=True)
        e = jnp.exp(g - m)
        w0 = e[:, 0:1] / (e[:, 0:1] + e[:, 1:2])
        o_ref[...] = w0 * ilv + (1.0 - w0) * blv

    in_specs = []
    args = []
    if chunked:
        in_specs += [pl.BlockSpec((4, _BLK, 64), lambda i: (0, i, 0)),
                     pl.BlockSpec((2, _BLK, _HW), lambda i: (0, i, 0))]
        args += [il4, cnt]
    else:
        in_specs.append(pl.BlockSpec((_BLK, _D), lambda i: (i, 0)))
        args.append(il)
    in_specs += [pl.BlockSpec((_BLK, _D), lambda i: (i, 0)),
                 pl.BlockSpec((_BLK, _D), lambda i: (i, 0)),
                 pl.BlockSpec((3 * _D, 2), lambda i: (0, 0)),
                 pl.BlockSpec((1, 2), lambda i: (0, 0))]
    args += [bl, feat, gate_W, gate_b.reshape(1, 2)]

    return pl.pallas_call(
        body,
        grid=(nr // _BLK,),
        in_specs=in_specs,
        out_specs=pl.BlockSpec((_BLK, _D), lambda i: (i, 0)),
        out_shape=jax.ShapeDtypeStruct((nr, _D), jnp.float32),
    )(*args)


def _pad_rows(x, nr):
    return jnp.pad(x, ((0, nr - x.shape[0]), (0, 0)))


def _level(srcg, dstb, deg2, featA, featB, W1, W2, emit_chunked):
    x = _pad_rows(jnp.concatenate([featA, featB], axis=0), _NR)
    a, hp0 = _prep_tc(deg2, x)
    s1 = _scatter_sc(hp0.reshape(4 * _NR, 64), srcg, dstb, _NR, 1)
    h1, hp1 = _layer1_tc(s1.reshape(4, _NR, 64), a, x, W1)
    s2 = _scatter_sc(hp1.reshape(4 * _NR, 64), srcg, dstb, _NR, 1)
    return _layer2_tc(s2.reshape(4, _NR, 64), a, h1, x, W2, emit_chunked)


def kernel(ui_edge_index, ub_edge_index, bi_edge_index, users_feature,
           items_feature, bundles_feature, W1_item, W2_item, W1_bundle,
           W2_bundle, gate_W, gate_b):
    offs4 = (jnp.arange(4, dtype=jnp.int32) * _NR)[:, None]

    def edges(ei):
        src = jnp.concatenate([ei[0], ei[1] + _U])
        dst = jnp.concatenate([ei[1] + _U, ei[0]])
        return (src[None, :] + offs4).reshape(-1, _EB), dst.reshape(-1, _EB)

    srcg_ui, dstb_ui = edges(ui_edge_index)
    srcg_ub, dstb_ub = edges(ub_edge_index)

    b_idx, i_idx = bi_edge_index[0], bi_edge_index[1]
    psrcg = ((i_idx + _U)[None, :] + offs4).reshape(-1, _EB)
    pdstb = b_idx.reshape(-1, _EB)

    degs, cnt = _hist_sc(dstb_ui, dstb_ub, pdstb)
    degs = degs.reshape(2, 2, _NR, _HW)
    cnt = cnt.reshape(2, _NRB, _HW)

    out_ui, ui_chunked = _level(srcg_ui, dstb_ui, degs[0], users_feature,
                                items_feature, W1_item, W2_item, True)
    (out_ub,) = _level(srcg_ub, dstb_ub, degs[1], users_feature,
                       bundles_feature, W1_bundle, W2_bundle, False)

    pooled4 = _scatter_sc(ui_chunked.reshape(4 * _NR, 64), psrcg, pdstb,
                          _NRB, 1)

    users_il = _pad_rows(out_ui[:_U], _NRB)
    users_bl = _pad_rows(out_ub[:_U], _NRB)
    bundles_bl = _pad_rows(out_ub[_U:_U + _B], _NRB)
    uf = _pad_rows(users_feature, _NRB)
    bf = _pad_rows(bundles_feature, _NRB)

    users_out = _gate_tc(users_il, None, None, users_bl, uf, gate_W, gate_b)
    bundles_out = _gate_tc(None, pooled4.reshape(4, _NRB, 64), cnt,
                           bundles_bl, bf, gate_W, gate_b)
    return jnp.concatenate([users_out[:_U], bundles_out[:_B]], axis=0)


# speedup vs baseline: 1.2718x; 1.0645x over previous
import functools

import jax
import jax.numpy as jnp
from jax import lax
from jax.experimental import pallas as pl
from jax.experimental.pallas import tpu as pltpu
from jax.experimental.pallas import tpu_sc as plsc

_U, _I, _B, _D = 10000, 10000, 10000, 256
_E = 160000
_EPS = 1e-8

_NR = 20480
_NRB = 10240
_BLK = 1024
_EB = 125
_HW = 16

_MESH = plsc.VectorSubcoreMesh(core_axis_name="c", subcore_axis_name="s")
_SC_PARAMS = pltpu.CompilerParams(use_tc_tiling_on_sc=False)


def _hist_sc(dstb, npad):
    nblk = dstb.shape[0]
    bpt = nblk // 32
    stripe = npad // 16

    @functools.partial(
        pl.kernel,
        out_type=jax.ShapeDtypeStruct((2 * npad, _HW), jnp.float32),
        mesh=_MESH,
        compiler_params=_SC_PARAMS,
        scratch_types=[
            pltpu.VMEM((bpt, _EB), jnp.int32),
            pltpu.VMEM((_EB, _HW), jnp.float32),
            pltpu.VMEM((128, _HW), jnp.float32),
            pltpu.VMEM_SHARED((npad, _HW), jnp.float32),
            pltpu.SemaphoreType.DMA,
        ],
    )
    def k(dstb_hbm, out_hbm, dstv, onesb, zbuf, acc, sem):
        cid = lax.axis_index("c")
        sid = lax.axis_index("s")
        wid = sid * 2 + cid
        ones16 = jnp.ones((16,), jnp.float32)
        zero16 = jnp.zeros((16,), jnp.float32)
        pltpu.sync_copy(dstb_hbm.at[pl.ds(wid * bpt, bpt)], dstv)

        def obody(i, _):
            onesb[i, pl.ds(0, 16)] = ones16
            return 0
        lax.fori_loop(0, _EB, obody, 0)

        def zbody(i, _):
            zbuf[i, pl.ds(0, 16)] = zero16
            return 0
        lax.fori_loop(0, 128, zbody, 0)

        for t in range(stripe // 128):
            pltpu.sync_copy(zbuf, acc.at[pl.ds(sid * stripe + t * 128, 128)])
        plsc.subcore_barrier()

        def body(n, _):
            for j in range(4):
                pltpu.async_copy(onesb, acc.at[dstv.at[n * 4 + j]], sem,
                                 add=True)
            for j in range(4):
                pltpu.make_async_copy(onesb, acc.at[dstv.at[0]], sem).wait()
            return 0
        lax.fori_loop(0, bpt // 4, body, 0)
        plsc.subcore_barrier()
        pltpu.sync_copy(acc.at[pl.ds(sid * stripe, stripe)],
                        out_hbm.at[pl.ds(cid * npad + sid * stripe, stripe)])

    return k(dstb)


def _scatter_sc(table, srcg, dstb, npad_out, n_lv):
    nblk = dstb.shape[0] // n_lv
    bpt = nblk // 16
    SG = 8
    sgrps = bpt // SG
    stripe = npad_out // 16
    zcop = stripe // 64

    @functools.partial(
        pl.kernel,
        out_type=jax.ShapeDtypeStruct((n_lv * 4 * npad_out, 64), jnp.float32),
        mesh=_MESH,
        compiler_params=_SC_PARAMS,
        scratch_types=[
            pltpu.VMEM((3, SG, _EB), jnp.int32),
            pltpu.VMEM((3, SG, _EB), jnp.int32),
            pltpu.VMEM((4, _EB, 64), jnp.float32),
            pltpu.VMEM((64, 64), jnp.float32),
            pltpu.VMEM_SHARED((npad_out, 64), jnp.float32),
            pltpu.SemaphoreType.DMA,
            pltpu.SemaphoreType.DMA,
            pltpu.SemaphoreType.DMA,
        ],
    )
    def k(tab_hbm, srcg_hbm, dstb_hbm, out_hbm,
          srcv, dstv, rowsb, zbuf, acc, sem_g, sem_s, sem_i):
        cid = lax.axis_index("c")
        sid = lax.axis_index("s")
        zero16 = jnp.zeros((16,), jnp.float32)

        def zbody(i, _):
            r = lax.shift_right_logical(i, 2)
            c = lax.bitwise_and(i, 3)
            zbuf[r, pl.ds(c * 16, 16)] = zero16
            return 0
        lax.fori_loop(0, 256, zbody, 0)

        for lv in range(n_lv):
            for kk in range(2):
                chunk = 2 * cid + kk
                sb0 = (4 * lv + chunk) * nblk + sid * bpt
                db0 = lv * nblk + sid * bpt

                def fire_is(s, par):
                    pltpu.async_copy(srcg_hbm.at[pl.ds(sb0 + s * SG, SG)],
                                     srcv.at[par], sem_i)
                    pltpu.async_copy(dstb_hbm.at[pl.ds(db0 + s * SG, SG)],
                                     dstv.at[par], sem_i)

                def drain_is(par):
                    for _ in range(2):
                        pltpu.make_async_copy(dstb_hbm.at[pl.ds(db0, SG)],
                                              dstv.at[par], sem_i).wait()

                def fire_g(par, r):
                    pltpu.async_copy(tab_hbm.at[srcv.at[par, r]],
                                     rowsb.at[r % 4], sem_g)

                def drain_g(r):
                    pltpu.make_async_copy(tab_hbm.at[srcv.at[0, 0]],
                                          rowsb.at[r % 4], sem_g).wait()

                def fire_s(par, r):
                    pltpu.async_copy(rowsb.at[r % 4],
                                     acc.at[dstv.at[par, r]],
                                     sem_s, add=True)

                def drain_s(r):
                    pltpu.make_async_copy(rowsb.at[r % 4],
                                          acc.at[dstv.at[0, 0]],
                                          sem_s).wait()

                def steady_rows(par, pp, first):
                    for r in range(SG):
                        if not first or r >= 4:
                            drain_s(r % 4)
                        fire_g(par, r)
                        if first and r < 2:
                            continue
                        if r < 2:
                            drain_g((r - 2) % 4)
                            fire_s(pp, SG + r - 2)
                        else:
                            drain_g(r - 2)
                            fire_s(par, r - 2)

                for t in range(zcop):
                    pltpu.sync_copy(
                        zbuf, acc.at[pl.ds(sid * stripe + t * 64, 64)])
                plsc.subcore_barrier()

                fire_is(0, 0)
                drain_is(0)
                fire_is(1, 1)
                steady_rows(0, 0, True)

                def body(s, _):
                    par = lax.rem(s, 3)
                    pp = lax.rem(s + 2, 3)
                    pn = lax.rem(s + 1, 3)
                    drain_is(par)
                    fire_is(s + 1, pn)
                    steady_rows(par, pp, False)
                    return 0
                lax.fori_loop(1, sgrps - 1, body, 0)

                pe = (sgrps - 1) % 3
                drain_is(pe)
                steady_rows(pe, (sgrps - 2) % 3, False)
                drain_g(2)
                fire_s(pe, SG - 2)
                drain_g(3)
                fire_s(pe, SG - 1)
                for r in range(4):
                    drain_s(r)

                plsc.subcore_barrier()
                pltpu.sync_copy(
                    acc.at[pl.ds(sid * stripe, stripe)],
                    out_hbm.at[pl.ds((4 * lv + chunk) * npad_out
                                     + sid * stripe, stripe)])
                plsc.subcore_barrier()

    return k(table, srcg, dstb)


def _prep_tc(deg2, x):
    def body(d_ref, x_ref, a_ref, hp_ref):
        d = d_ref[0, :, 0:1] + d_ref[1, :, 0:1] + 1.0
        a = 1.0 / (jnp.sqrt(d) + _EPS)
        a_ref[...] = a
        hp = a * x_ref[...]
        for c in range(4):
            hp_ref[c] = hp[:, c * 64:(c + 1) * 64]

    return pl.pallas_call(
        body,
        grid=(_NR // _BLK,),
        in_specs=[pl.BlockSpec((2, _BLK, _HW), lambda i: (0, i, 0)),
                  pl.BlockSpec((_BLK, _D), lambda i: (i, 0))],
        out_specs=[pl.BlockSpec((_BLK, 1), lambda i: (i, 0)),
                   pl.BlockSpec((4, _BLK, 64), lambda i: (0, i, 0))],
        out_shape=[jax.ShapeDtypeStruct((_NR, 1), jnp.float32),
                   jax.ShapeDtypeStruct((4, _NR, 64), jnp.float32)],
    )(deg2, x)


def _layer1_tc(s4, a, x, W):
    def body(s_ref, a_ref, h_ref, w_ref, h1_ref, hp_ref):
        aa = a_ref[...]
        s = jnp.concatenate([s_ref[c] for c in range(4)], axis=1)
        m = aa * (s + aa * h_ref[...])
        h1 = jnp.tanh(jnp.dot(m, w_ref[...],
                              preferred_element_type=jnp.float32))
        h1_ref[...] = h1
        hp = aa * h1
        for c in range(4):
            hp_ref[c] = hp[:, c * 64:(c + 1) * 64]

    return pl.pallas_call(
        body,
        grid=(_NR // _BLK,),
        in_specs=[pl.BlockSpec((4, _BLK, 64), lambda i: (0, i, 0)),
                  pl.BlockSpec((_BLK, 1), lambda i: (i, 0)),
                  pl.BlockSpec((_BLK, _D), lambda i: (i, 0)),
                  pl.BlockSpec((_D, _D), lambda i: (0, 0))],
        out_specs=[pl.BlockSpec((_BLK, _D), lambda i: (i, 0)),
                   pl.BlockSpec((4, _BLK, 64), lambda i: (0, i, 0))],
        out_shape=[jax.ShapeDtypeStruct((_NR, _D), jnp.float32),
                   jax.ShapeDtypeStruct((4, _NR, 64), jnp.float32)],
    )(s4, a, x, W)


def _layer2_tc(s4, a, h1, x, W, emit_chunked):
    def body(*refs):
        if emit_chunked:
            s_ref, a_ref, h1_ref, x_ref, w_ref, o_ref, oc_ref = refs
        else:
            s_ref, a_ref, h1_ref, x_ref, w_ref, o_ref = refs
        aa = a_ref[...]
        h1v = h1_ref[...]
        s = jnp.concatenate([s_ref[c] for c in range(4)], axis=1)
        m = aa * (s + aa * h1v)
        h2 = jnp.tanh(jnp.dot(m, w_ref[...],
                              preferred_element_type=jnp.float32))
        o = (x_ref[...] + h1v + h2) * (1.0 / 3.0)
        o_ref[...] = o
        if emit_chunked:
            for c in range(4):
                oc_ref[c] = o[:, c * 64:(c + 1) * 64]

    out_specs = [pl.BlockSpec((_BLK, _D), lambda i: (i, 0))]
    out_shape = [jax.ShapeDtypeStruct((_NR, _D), jnp.float32)]
    if emit_chunked:
        out_specs.append(pl.BlockSpec((4, _BLK, 64), lambda i: (0, i, 0)))
        out_shape.append(jax.ShapeDtypeStruct((4, _NR, 64), jnp.float32))

    return pl.pallas_call(
        body,
        grid=(_NR // _BLK,),
        in_specs=[pl.BlockSpec((4, _BLK, 64), lambda i: (0, i, 0)),
                  pl.BlockSpec((_BLK, 1), lambda i: (i, 0)),
                  pl.BlockSpec((_BLK, _D), lambda i: (i, 0)),
                  pl.BlockSpec((_BLK, _D), lambda i: (i, 0)),
                  pl.BlockSpec((_D, _D), lambda i: (0, 0))],
        out_specs=out_specs,
        out_shape=out_shape,
    )(s4, a, h1, x, W)


_GB = 1000


def _gate_tc(il, il4, cnt, bl, bl_off, feat, gate_W, gate_b):
    chunked = il4 is not None

    def body(*refs):
        if chunked:
            il_ref, cnt_ref, bl_ref, f_ref, w_ref, b_ref, o_ref = refs
            cntv = cnt_ref[0, :, 0:1] + cnt_ref[1, :, 0:1]
            ilv = jnp.concatenate([il_ref[c] for c in range(4)], axis=1)
            ilv = ilv / (cntv + _EPS)
        else:
            il_ref, bl_ref, f_ref, w_ref, b_ref, o_ref = refs
            ilv = il_ref[...]
        blv = bl_ref[...]
        g = (jnp.dot(ilv, w_ref[0:_D, :], preferred_element_type=jnp.float32)
             + jnp.dot(blv, w_ref[_D:2 * _D, :],
                       preferred_element_type=jnp.float32)
             + jnp.dot(f_ref[...], w_ref[2 * _D:3 * _D, :],
                       preferred_element_type=jnp.float32)
             + b_ref[...])
        m = jnp.max(g, axis=1, keepdims=True)
        e = jnp.exp(g - m)
        w0 = e[:, 0:1] / (e[:, 0:1] + e[:, 1:2])
        o_ref[...] = w0 * ilv + (1.0 - w0) * blv

    in_specs = []
    args = []
    if chunked:
        in_specs += [pl.BlockSpec((4, _GB, 64), lambda i: (0, i, 0)),
                     pl.BlockSpec((2, _GB, _HW), lambda i: (0, i, 0))]
        args += [il4, cnt]
    else:
        in_specs.append(pl.BlockSpec((_GB, _D), lambda i: (i, 0)))
        args.append(il)
    in_specs += [pl.BlockSpec((_GB, _D), lambda i, o=bl_off: (i + o, 0)),
                 pl.BlockSpec((_GB, _D), lambda i: (i, 0)),
                 pl.BlockSpec((3 * _D, 2), lambda i: (0, 0)),
                 pl.BlockSpec((1, 2), lambda i: (0, 0))]
    args += [bl, feat, gate_W, gate_b.reshape(1, 2)]

    return pl.pallas_call(
        body,
        grid=(_U // _GB,),
        in_specs=in_specs,
        out_specs=pl.BlockSpec((_GB, _D), lambda i: (i, 0)),
        out_shape=jax.ShapeDtypeStruct((_U, _D), jnp.float32),
    )(*args)


def _pad_rows(x, nr):
    return jnp.pad(x, ((0, nr - x.shape[0]), (0, 0)))


def _level(srcg, dstb, featA, featB, W1, W2, emit_chunked):
    deg2 = _hist_sc(dstb, _NR).reshape(2, _NR, _HW)
    x = _pad_rows(jnp.concatenate([featA, featB], axis=0), _NR)
    a, hp0 = _prep_tc(deg2, x)
    s1 = _scatter_sc(hp0.reshape(4 * _NR, 64), srcg, dstb, _NR, 1)
    h1, hp1 = _layer1_tc(s1.reshape(4, _NR, 64), a, x, W1)
    s2 = _scatter_sc(hp1.reshape(4 * _NR, 64), srcg, dstb, _NR, 1)
    return _layer2_tc(s2.reshape(4, _NR, 64), a, h1, x, W2, emit_chunked)


def kernel(ui_edge_index, ub_edge_index, bi_edge_index, users_feature,
           items_feature, bundles_feature, W1_item, W2_item, W1_bundle,
           W2_bundle, gate_W, gate_b):
    offs4 = (jnp.arange(4, dtype=jnp.int32) * _NR)[:, None]

    def edges(ei):
        src = jnp.concatenate([ei[0], ei[1] + _U])
        dst = jnp.concatenate([ei[1] + _U, ei[0]])
        return (src[None, :] + offs4).reshape(-1, _EB), dst.reshape(-1, _EB)

    srcg_ui, dstb_ui = edges(ui_edge_index)
    srcg_ub, dstb_ub = edges(ub_edge_index)

    b_idx, i_idx = bi_edge_index[0], bi_edge_index[1]
    psrcg = ((i_idx + _U)[None, :] + offs4).reshape(-1, _EB)
    pdstb = b_idx.reshape(-1, _EB)
    cnt = _hist_sc(pdstb, _NRB).reshape(2, _NRB, _HW)

    out_ui, ui_chunked = _level(srcg_ui, dstb_ui, users_feature,
                                items_feature, W1_item, W2_item, True)
    (out_ub,) = _level(srcg_ub, dstb_ub, users_feature,
                       bundles_feature, W1_bundle, W2_bundle, False)

    pooled4 = _scatter_sc(ui_chunked.reshape(4 * _NR, 64), psrcg, pdstb,
                          _NRB, 1)

    users_out = _gate_tc(out_ui, None, None, out_ub, 0, users_feature,
                         gate_W, gate_b)
    bundles_out = _gate_tc(None, pooled4.reshape(4, _NRB, 64), cnt,
                           out_ub, 10, bundles_feature, gate_W, gate_b)
    return jnp.concatenate([users_out, bundles_out], axis=0)
